# baseline (speedup 1.0000x reference)
import functools

import jax
import jax.numpy as jnp
from jax import lax
from jax.experimental import pallas as pl
from jax.experimental.pallas import tpu as pltpu

_S = 512
_D = 512
_MAX_REL = 32
_VOCAB = 2 * _MAX_REL + 1
_KPAD = 128
_NBUF = 3


def _rpe_kernel(x_hbm, pe_ref, out_hbm, rowpe_ref, bufs, si0, si1, si2, so0, so1, so2):
    sins = [si0, si1, si2]
    souts = [so0, so1, so2]
    nb = x_hbm.shape[0]

    i = lax.broadcasted_iota(jnp.int32, (_S, _KPAD), 0)
    k = lax.broadcasted_iota(jnp.int32, (_S, _KPAD), 1)
    r = k - _MAX_REL
    mid = ((k >= 1) & (k <= _VOCAB - 2) & (r <= i) & (r >= i - (_S - 1)))
    counts = mid.astype(jnp.float32)
    left = jnp.maximum(_S - _MAX_REL - i, 0).astype(jnp.float32)
    right = jnp.maximum(i - _MAX_REL + 1, 0).astype(jnp.float32)
    counts = counts + jnp.where(k == 0, left, 0.0)
    counts = counts + jnp.where(k == _VOCAB - 1, right, 0.0)
    rowpe = jnp.dot(
        counts, pe_ref[...], preferred_element_type=jnp.float32
    ) * (1.0 / _S)
    rowpe_ref[...] = rowpe

    def cin(b):
        return pltpu.make_async_copy(
            x_hbm.at[b], bufs.at[b % _NBUF], sins[b % _NBUF])

    def cout(b):
        return pltpu.make_async_copy(
            bufs.at[b % _NBUF], out_hbm.at[b], souts[b % _NBUF])

    cin(0).start()
    if nb > 1:
        cin(1).start()
    for b in range(nb):
        if 1 <= b and b + 1 < nb:
            if b >= 2:
                cout(b - 2).wait()
            cin(b + 1).start()
        cin(b).wait()
        bufs[b % _NBUF] = bufs[b % _NBUF] + rowpe_ref[...]
        cout(b).start()
    for b in range(max(0, nb - _NBUF), nb):
        cout(b).wait()


@jax.jit
def kernel(x, pe):
    b, s, d = x.shape
    pe_padded = jnp.zeros((_KPAD, d), dtype=pe.dtype).at[: pe.shape[0]].set(pe)
    return pl.pallas_call(
        _rpe_kernel,
        in_specs=[
            pl.BlockSpec(memory_space=pl.ANY),
            pl.BlockSpec(memory_space=pltpu.VMEM),
        ],
        out_specs=pl.BlockSpec(memory_space=pl.ANY),
        out_shape=jax.ShapeDtypeStruct((b, s, d), x.dtype),
        scratch_shapes=[
            pltpu.VMEM((s, d), jnp.float32),
            pltpu.VMEM((_NBUF, s, d), jnp.float32),
            pltpu.SemaphoreType.DMA,
            pltpu.SemaphoreType.DMA,
            pltpu.SemaphoreType.DMA,
            pltpu.SemaphoreType.DMA,
            pltpu.SemaphoreType.DMA,
            pltpu.SemaphoreType.DMA,
        ],
    )(x, pe_padded)
